# Initial kernel scaffold; baseline (speedup 1.0000x reference)
import jax
import jax.numpy as jnp
from jax.experimental import pallas as pl


def kernel(z, weight):
    raise NotImplementedError("write your pallas kernel here")


# speedup vs baseline: 2.1095x; 2.1095x over previous
import functools

import jax
import jax.numpy as jnp
from jax import lax
from jax.experimental import pallas as pl

N_CODES = 1024
DIM = 256
N_STAGES = 4
BETA_C = 0.25
BL = 512


def _vq_body(nsteps, z_ref, w_ref, zq_ref, enc_ref, idx_ref, loss_ref, ppl_ref,
             cnt_scr):
    i = pl.program_id(0)

    @pl.when(i == 0)
    def _init():
        loss_ref[...] = jnp.zeros_like(loss_ref)
        cnt_scr[...] = jnp.zeros_like(cnt_scr)

    w = w_ref[...]
    wsq = lax.dot_general(jnp.ones((1, DIM), jnp.float32), w * w,
                          (((1,), (1,)), ((), ())),
                          preferred_element_type=jnp.float32,
                          precision=lax.Precision.HIGHEST)
    w1 = w.astype(jnp.bfloat16)
    w2 = (w - w1.astype(jnp.float32)).astype(jnp.bfloat16)
    w3 = (w - w1.astype(jnp.float32) - w2.astype(jnp.float32)).astype(jnp.bfloat16)

    residual = z_ref[...]
    qsum = jnp.zeros((BL, DIM), jnp.float32)
    lsum = jnp.zeros((1, 1), jnp.float32)
    cnt = jnp.zeros((1, N_CODES), jnp.float32)
    idx_cols = []
    iota = lax.broadcasted_iota(jnp.int32, (BL, N_CODES), 1)
    for q in range(N_STAGES):
        s = lax.dot_general(residual, w, (((1,), (1,)), ((), ())),
                            preferred_element_type=jnp.float32)
        rsq = jnp.sum(residual * residual, axis=1, keepdims=True)
        dist = (rsq + wsq) - 2.0 * s
        dmin = jnp.min(dist, axis=1, keepdims=True)
        idxm = jnp.min(jnp.where(dist <= dmin, iota, N_CODES), axis=1,
                       keepdims=True)
        oh = (iota == idxm).astype(jnp.float32)
        enc_ref[q] = oh
        cnt = cnt + jnp.sum(oh, axis=0, keepdims=True)
        ohb = oh.astype(jnp.bfloat16)
        zqd = (lax.dot_general(ohb, w1, (((1,), (0,)), ((), ())),
                               preferred_element_type=jnp.float32)
               + lax.dot_general(ohb, w2, (((1,), (0,)), ((), ())),
                                 preferred_element_type=jnp.float32)
               + lax.dot_general(ohb, w3, (((1,), (0,)), ((), ())),
                                 preferred_element_type=jnp.float32))
        qsum = qsum + zqd
        residual = residual - zqd
        r2 = jnp.sum(residual * residual, axis=1, keepdims=True)
        lsum = lsum + jnp.sum(r2, axis=0, keepdims=True)
        idx_cols.append(idxm)

    zq_ref[...] = qsum
    idx_ref[...] = jnp.concatenate(idx_cols, axis=1)
    loss_ref[...] += lsum
    cnt_scr[...] += cnt

    @pl.when(i == nsteps - 1)
    def _fini():
        loss_ref[...] = loss_ref[...] * (BETA_C / (nsteps * BL * DIM))
        avg = cnt_scr[...] * (1.0 / (nsteps * BL * N_STAGES))
        ent = jnp.sum(avg * jnp.log(avg + 1e-10), axis=1, keepdims=True)
        ppl_ref[...] = jnp.exp(-ent)


@jax.jit
def kernel(z, weight):
    b, c, h, w = z.shape
    rows = b * h * w
    nsteps = rows // BL
    z_flat = jnp.transpose(z, (0, 2, 3, 1)).reshape(rows, DIM)

    from jax.experimental.pallas import tpu as pltpu
    zq_flat, enc, idx, loss, ppl = pl.pallas_call(
        functools.partial(_vq_body, nsteps),
        grid=(nsteps,),
        in_specs=[
            pl.BlockSpec((BL, DIM), lambda i: (i, 0)),
            pl.BlockSpec((N_CODES, DIM), lambda i: (0, 0)),
        ],
        out_specs=[
            pl.BlockSpec((BL, DIM), lambda i: (i, 0)),
            pl.BlockSpec((N_STAGES, BL, N_CODES), lambda i: (0, i, 0)),
            pl.BlockSpec((BL, N_STAGES), lambda i: (i, 0)),
            pl.BlockSpec((1, 1), lambda i: (0, 0)),
            pl.BlockSpec((1, 1), lambda i: (0, 0)),
        ],
        out_shape=[
            jax.ShapeDtypeStruct((rows, DIM), jnp.float32),
            jax.ShapeDtypeStruct((N_STAGES, rows, N_CODES), jnp.float32),
            jax.ShapeDtypeStruct((rows, N_STAGES), jnp.int32),
            jax.ShapeDtypeStruct((1, 1), jnp.float32),
            jax.ShapeDtypeStruct((1, 1), jnp.float32),
        ],
        scratch_shapes=[pltpu.VMEM((1, N_CODES), jnp.float32)],
    )(z_flat, weight)

    z_q = jnp.transpose(zq_flat.reshape(b, h, w, DIM), (0, 3, 1, 2))
    encodings_cat = enc.reshape(N_STAGES * rows, N_CODES)
    indices_stack = jnp.transpose(idx.reshape(b, h, w, N_STAGES), (0, 3, 1, 2))
    return (z_q, loss[0, 0], ppl[0, 0], encodings_cat, indices_stack)
